# Initial kernel scaffold; baseline (speedup 1.0000x reference)
import jax
import jax.numpy as jnp
from jax.experimental import pallas as pl


def kernel(x, edge_index, Wl0, bl0, Wr0, Wl1, bl1, Wr1, Wl2, bl2, Wr2, Wl3, bl3, Wr3):
    raise NotImplementedError("write your pallas kernel here")


# speedup vs baseline: 7.9572x; 7.9572x over previous
import jax
import jax.numpy as jnp
from jax import lax
from jax.experimental import pallas as pl
from jax.experimental.pallas import tpu as pltpu
from jax.experimental.pallas import tpu_sc as plsc

N = 10000
E = 320000
D = 32
NC = 2
NS = 16
NW = NC * NS
C = 128
NCH = 80
EPW = NCH * C
EPAD = EPW * NW
RPT = (-(-(N + 1) // NS) + 7) // 8 * 8
NA = RPT * NS
CW = 16


def _sc_scatter(with_cnt):
    mesh = plsc.VectorSubcoreMesh(core_axis_name="c", subcore_axis_name="s")
    out_type = [jax.ShapeDtypeStruct((NC * NA, D), jnp.float32)]
    scratch = [
        pltpu.VMEM((NCH, C), jnp.int32),
        pltpu.VMEM((NCH, C), jnp.int32),
        pltpu.VMEM((C, D), jnp.float32),
        pltpu.VMEM_SHARED((NA, D), jnp.float32),
        pltpu.SemaphoreType.DMA,
    ]
    if with_cnt:
        out_type.append(jax.ShapeDtypeStruct((NC * NA, CW), jnp.float32))
        scratch += [
            pltpu.VMEM((C, CW), jnp.float32),
            pltpu.VMEM_SHARED((NA, CW), jnp.float32),
        ]

    def body(xl_hbm, src_hbm, dst_hbm, z_hbm, *rest):
        if with_cnt:
            (zc_hbm, ones_hbm, s_out, cnt_out,
             src_v, dst_v, rows_v, acc_sh, sem, ones_v, cnt_sh) = rest
        else:
            (s_out, src_v, dst_v, rows_v, acc_sh, sem) = rest
        c = lax.axis_index("c")
        s = lax.axis_index("s")
        wid = s * NC + c
        row0 = s * RPT
        pltpu.sync_copy(src_hbm.at[pl.ds(wid * NCH, NCH)], src_v)
        pltpu.sync_copy(dst_hbm.at[pl.ds(wid * NCH, NCH)], dst_v)
        pltpu.sync_copy(z_hbm, acc_sh.at[pl.ds(row0, RPT)])
        if with_cnt:
            pltpu.sync_copy(zc_hbm, cnt_sh.at[pl.ds(row0, RPT)])
            pltpu.sync_copy(ones_hbm, ones_v)
        plsc.subcore_barrier()

        @pl.loop(0, NCH)
        def _edge_chunk(j):
            pltpu.async_copy(xl_hbm.at[src_v.at[j]], rows_v, sem).wait()
            pltpu.sync_copy(rows_v, acc_sh.at[dst_v.at[j]], add=True)
            if with_cnt:
                pltpu.sync_copy(ones_v, cnt_sh.at[dst_v.at[j]], add=True)

        plsc.subcore_barrier()
        pltpu.sync_copy(acc_sh.at[pl.ds(row0, RPT)],
                        s_out.at[pl.ds(c * NA + row0, RPT)])
        if with_cnt:
            pltpu.sync_copy(cnt_sh.at[pl.ds(row0, RPT)],
                            cnt_out.at[pl.ds(c * NA + row0, RPT)])

    return pl.kernel(body, out_type=tuple(out_type), mesh=mesh,
                     scratch_types=scratch,
                     compiler_params=pltpu.CompilerParams(
                         use_tc_tiling_on_sc=False))


def _tc_proj(x_ref, wl_ref, wr_ref, b_ref, xl_ref, r_ref):
    x = x_ref[...]
    xl_ref[:N, :] = jnp.dot(x, wl_ref[...], preferred_element_type=jnp.float32)
    xl_ref[N:, :] = jnp.zeros((NA - N, D), jnp.float32)
    r_ref[...] = (jnp.dot(x, wr_ref[...], preferred_element_type=jnp.float32)
                  + b_ref[...])


def _tc_first(s_ref, cnt_ref, r_ref, wl_ref, wr_ref, b_ref,
              h_ref, xl_ref, rn_ref, inv_ref):
    cnt = cnt_ref[:N, 0:1] + cnt_ref[NA:NA + N, 0:1]
    inv = 1.0 / jnp.maximum(cnt, 1.0)
    inv_ref[...] = inv
    ssum = s_ref[:N, :] + s_ref[NA:NA + N, :]
    h = jnp.maximum(ssum * inv + r_ref[...], 0.0)
    h_ref[...] = h
    xl_ref[:N, :] = jnp.dot(h, wl_ref[...], preferred_element_type=jnp.float32)
    xl_ref[N:, :] = jnp.zeros((NA - N, D), jnp.float32)
    rn_ref[...] = (jnp.dot(h, wr_ref[...], preferred_element_type=jnp.float32)
                   + b_ref[...])


def _tc_mid(s_ref, inv_ref, r_ref, res_ref, wl_ref, wr_ref, b_ref,
            h_ref, xl_ref, rn_ref):
    ssum = s_ref[:N, :] + s_ref[NA:NA + N, :]
    h = jnp.maximum(ssum * inv_ref[...] + r_ref[...] + res_ref[...], 0.0)
    h_ref[...] = h
    xl_ref[:N, :] = jnp.dot(h, wl_ref[...], preferred_element_type=jnp.float32)
    xl_ref[N:, :] = jnp.zeros((NA - N, D), jnp.float32)
    rn_ref[...] = (jnp.dot(h, wr_ref[...], preferred_element_type=jnp.float32)
                   + b_ref[...])


def _tc_final(s_ref, inv_ref, r_ref, out_ref):
    ssum = s_ref[:N, :] + s_ref[NA:NA + N, :]
    out_ref[...] = ssum * inv_ref[...] + r_ref[...]


def _call(body, out_shapes, *args):
    return pl.pallas_call(body, out_shape=out_shapes)(*args)


def kernel(x, edge_index, Wl0, bl0, Wr0, Wl1, bl1, Wr1, Wl2, bl2, Wr2,
           Wl3, bl3, Wr3):
    f32 = jnp.float32
    src = edge_index[0]
    dst = edge_index[1]
    pad = EPAD - E
    src_p = jnp.concatenate([src, jnp.zeros((pad,), jnp.int32)])
    dst_p = jnp.concatenate([dst, jnp.full((pad,), N, jnp.int32)])
    src_p = src_p.reshape(NW * NCH, C)
    dst_p = dst_p.reshape(NW * NCH, C)
    z32 = jnp.zeros((RPT, D), f32)
    z16 = jnp.zeros((RPT, CW), f32)
    ones = jnp.ones((C, CW), f32)
    mk = jax.ShapeDtypeStruct

    sc_cnt = _sc_scatter(True)
    sc = _sc_scatter(False)

    xl0, r0 = _call(_tc_proj, (mk((NA, D), f32), mk((N, D), f32)),
                    x, Wl0.T, Wr0.T, bl0.reshape(1, D))
    s0, cnt = sc_cnt(xl0, src_p, dst_p, z32, z16, ones)
    h0, xl1, r1, inv = _call(
        _tc_first, (mk((N, D), f32), mk((NA, D), f32), mk((N, D), f32),
                    mk((N, 1), f32)),
        s0, cnt, r0, Wl1.T, Wr1.T, bl1.reshape(1, D))
    (s1,) = sc(xl1, src_p, dst_p, z32)
    h1, xl2, r2 = _call(
        _tc_mid, (mk((N, D), f32), mk((NA, D), f32), mk((N, D), f32)),
        s1, inv, r1, h0, Wl2.T, Wr2.T, bl2.reshape(1, D))
    (s2,) = sc(xl2, src_p, dst_p, z32)
    h2, xl3, r3 = _call(
        _tc_mid, (mk((N, D), f32), mk((NA, D), f32), mk((N, D), f32)),
        s2, inv, r2, h1, Wl3.T, Wr3.T, bl3.reshape(1, D))
    (s3,) = sc(xl3, src_p, dst_p, z32)
    out = _call(_tc_final, mk((N, D), f32), s3, inv, r3)
    return out


# speedup vs baseline: 8.5370x; 1.0729x over previous
import jax
import jax.numpy as jnp
from jax import lax
from jax.experimental import pallas as pl
from jax.experimental.pallas import tpu as pltpu
from jax.experimental.pallas import tpu_sc as plsc

N = 10000
E = 320000
D = 32
NC = 2
NS = 16
NW = NC * NS
C = 128
NCH = 80
EPW = NCH * C
EPAD = EPW * NW
RPT = (-(-(N + 1) // NS) + 7) // 8 * 8
NA = RPT * NS
CW = 16


def _sc_scatter(with_cnt):
    mesh = plsc.VectorSubcoreMesh(core_axis_name="c", subcore_axis_name="s")
    out_type = [jax.ShapeDtypeStruct((NC * NA, D), jnp.float32)]
    scratch = [
        pltpu.VMEM((NCH, C), jnp.int32),
        pltpu.VMEM((NCH, C), jnp.int32),
        pltpu.VMEM((2, C, D), jnp.float32),
        pltpu.VMEM_SHARED((NA, D), jnp.float32),
        pltpu.SemaphoreType.DMA,
    ]
    if with_cnt:
        out_type.append(jax.ShapeDtypeStruct((NC * NA, CW), jnp.float32))
        scratch += [
            pltpu.VMEM((C, CW), jnp.float32),
            pltpu.VMEM_SHARED((NA, CW), jnp.float32),
        ]

    def body(xl_hbm, src_hbm, dst_hbm, z_hbm, *rest):
        if with_cnt:
            (zc_hbm, ones_hbm, s_out, cnt_out,
             src_v, dst_v, rows_v, acc_sh, sem, ones_v, cnt_sh) = rest
        else:
            (s_out, src_v, dst_v, rows_v, acc_sh, sem) = rest
        c = lax.axis_index("c")
        s = lax.axis_index("s")
        wid = s * NC + c
        row0 = s * RPT
        pltpu.sync_copy(src_hbm.at[pl.ds(wid * NCH, NCH)], src_v)
        pltpu.sync_copy(dst_hbm.at[pl.ds(wid * NCH, NCH)], dst_v)
        pltpu.sync_copy(z_hbm, acc_sh.at[pl.ds(row0, RPT)])
        if with_cnt:
            pltpu.sync_copy(zc_hbm, cnt_sh.at[pl.ds(row0, RPT)])
            pltpu.sync_copy(ones_hbm, ones_v)
        plsc.subcore_barrier()

        pltpu.async_copy(xl_hbm.at[src_v.at[0]], rows_v.at[0], sem)

        @pl.loop(0, NCH, step=2)
        def _edge_chunk(j):
            for b in range(2):
                pltpu.make_async_copy(xl_hbm.at[pl.ds(0, C)],
                                      rows_v.at[b], sem).wait()

                @pl.when(j + b + 1 < NCH)
                def _():
                    pltpu.async_copy(xl_hbm.at[src_v.at[j + b + 1]],
                                     rows_v.at[1 - b], sem)

                pltpu.sync_copy(rows_v.at[b], acc_sh.at[dst_v.at[j + b]],
                                add=True)
                if with_cnt:
                    pltpu.sync_copy(ones_v, cnt_sh.at[dst_v.at[j + b]],
                                    add=True)

        plsc.subcore_barrier()
        pltpu.sync_copy(acc_sh.at[pl.ds(row0, RPT)],
                        s_out.at[pl.ds(c * NA + row0, RPT)])
        if with_cnt:
            pltpu.sync_copy(cnt_sh.at[pl.ds(row0, RPT)],
                            cnt_out.at[pl.ds(c * NA + row0, RPT)])

    return pl.kernel(body, out_type=tuple(out_type), mesh=mesh,
                     scratch_types=scratch,
                     compiler_params=pltpu.CompilerParams(
                         use_tc_tiling_on_sc=False))


def _tc_proj(x_ref, wl_ref, wr_ref, b_ref, xl_ref, r_ref):
    x = x_ref[...]
    xl_ref[:N, :] = jnp.dot(x, wl_ref[...], preferred_element_type=jnp.float32)
    xl_ref[N:, :] = jnp.zeros((NA - N, D), jnp.float32)
    r_ref[...] = (jnp.dot(x, wr_ref[...], preferred_element_type=jnp.float32)
                  + b_ref[...])


def _tc_first(s_ref, cnt_ref, r_ref, wl_ref, wr_ref, b_ref,
              h_ref, xl_ref, rn_ref, inv_ref):
    cnt = cnt_ref[:N, 0:1] + cnt_ref[NA:NA + N, 0:1]
    inv = 1.0 / jnp.maximum(cnt, 1.0)
    inv_ref[...] = inv
    ssum = s_ref[:N, :] + s_ref[NA:NA + N, :]
    h = jnp.maximum(ssum * inv + r_ref[...], 0.0)
    h_ref[...] = h
    xl_ref[:N, :] = jnp.dot(h, wl_ref[...], preferred_element_type=jnp.float32)
    xl_ref[N:, :] = jnp.zeros((NA - N, D), jnp.float32)
    rn_ref[...] = (jnp.dot(h, wr_ref[...], preferred_element_type=jnp.float32)
                   + b_ref[...])


def _tc_mid(s_ref, inv_ref, r_ref, res_ref, wl_ref, wr_ref, b_ref,
            h_ref, xl_ref, rn_ref):
    ssum = s_ref[:N, :] + s_ref[NA:NA + N, :]
    h = jnp.maximum(ssum * inv_ref[...] + r_ref[...] + res_ref[...], 0.0)
    h_ref[...] = h
    xl_ref[:N, :] = jnp.dot(h, wl_ref[...], preferred_element_type=jnp.float32)
    xl_ref[N:, :] = jnp.zeros((NA - N, D), jnp.float32)
    rn_ref[...] = (jnp.dot(h, wr_ref[...], preferred_element_type=jnp.float32)
                   + b_ref[...])


def _tc_final(s_ref, inv_ref, r_ref, out_ref):
    ssum = s_ref[:N, :] + s_ref[NA:NA + N, :]
    out_ref[...] = ssum * inv_ref[...] + r_ref[...]


def _call(body, out_shapes, *args):
    return pl.pallas_call(body, out_shape=out_shapes)(*args)


def kernel(x, edge_index, Wl0, bl0, Wr0, Wl1, bl1, Wr1, Wl2, bl2, Wr2,
           Wl3, bl3, Wr3):
    f32 = jnp.float32
    src = edge_index[0]
    dst = edge_index[1]
    pad = EPAD - E
    src_p = jnp.concatenate([src, jnp.zeros((pad,), jnp.int32)])
    dst_p = jnp.concatenate([dst, jnp.full((pad,), N, jnp.int32)])
    src_p = src_p.reshape(NW * NCH, C)
    dst_p = dst_p.reshape(NW * NCH, C)
    z32 = jnp.zeros((RPT, D), f32)
    z16 = jnp.zeros((RPT, CW), f32)
    ones = jnp.ones((C, CW), f32)
    mk = jax.ShapeDtypeStruct

    sc_cnt = _sc_scatter(True)
    sc = _sc_scatter(False)

    xl0, r0 = _call(_tc_proj, (mk((NA, D), f32), mk((N, D), f32)),
                    x, Wl0.T, Wr0.T, bl0.reshape(1, D))
    s0, cnt = sc_cnt(xl0, src_p, dst_p, z32, z16, ones)
    h0, xl1, r1, inv = _call(
        _tc_first, (mk((N, D), f32), mk((NA, D), f32), mk((N, D), f32),
                    mk((N, 1), f32)),
        s0, cnt, r0, Wl1.T, Wr1.T, bl1.reshape(1, D))
    (s1,) = sc(xl1, src_p, dst_p, z32)
    h1, xl2, r2 = _call(
        _tc_mid, (mk((N, D), f32), mk((NA, D), f32), mk((N, D), f32)),
        s1, inv, r1, h0, Wl2.T, Wr2.T, bl2.reshape(1, D))
    (s2,) = sc(xl2, src_p, dst_p, z32)
    h2, xl3, r3 = _call(
        _tc_mid, (mk((N, D), f32), mk((NA, D), f32), mk((N, D), f32)),
        s2, inv, r2, h1, Wl3.T, Wr3.T, bl3.reshape(1, D))
    (s3,) = sc(xl3, src_p, dst_p, z32)
    out = _call(_tc_final, mk((N, D), f32), s3, inv, r3)
    return out


# speedup vs baseline: 10.2721x; 1.2032x over previous
import jax
import jax.numpy as jnp
from jax import lax
from jax.experimental import pallas as pl
from jax.experimental.pallas import tpu as pltpu
from jax.experimental.pallas import tpu_sc as plsc

N = 10000
E = 320000
D = 32
NC = 2
NS = 16
NW = NC * NS
C = 128
NCH = 80
NB = 8
EPW = NCH * C
EPAD = EPW * NW
RPT = (-(-(N + 1) // NS) + 7) // 8 * 8
NA = RPT * NS
CW = 16


def _sc_scatter(with_cnt):
    mesh = plsc.VectorSubcoreMesh(core_axis_name="c", subcore_axis_name="s")
    out_type = [jax.ShapeDtypeStruct((NC * NA, D), jnp.float32)]
    scratch = [
        pltpu.VMEM((NCH, C), jnp.int32),
        pltpu.VMEM((NCH, C), jnp.int32),
        pltpu.VMEM((NB, C, D), jnp.float32),
        pltpu.VMEM_SHARED((NA, D), jnp.float32),
        pltpu.SemaphoreType.DMA,
    ]
    if with_cnt:
        out_type.append(jax.ShapeDtypeStruct((NC * NA, CW), jnp.float32))
        scratch += [
            pltpu.VMEM((C, CW), jnp.float32),
            pltpu.VMEM_SHARED((NA, CW), jnp.float32),
        ]

    def body(xl_hbm, src_hbm, dst_hbm, z_hbm, *rest):
        if with_cnt:
            (zc_hbm, ones_hbm, s_out, cnt_out,
             src_v, dst_v, rows_v, acc_sh, sem, ones_v, cnt_sh) = rest
        else:
            (s_out, src_v, dst_v, rows_v, acc_sh, sem) = rest
        c = lax.axis_index("c")
        s = lax.axis_index("s")
        wid = s * NC + c
        row0 = s * RPT
        pltpu.sync_copy(src_hbm.at[pl.ds(wid * NCH, NCH)], src_v)
        pltpu.sync_copy(dst_hbm.at[pl.ds(wid * NCH, NCH)], dst_v)
        pltpu.sync_copy(z_hbm, acc_sh.at[pl.ds(row0, RPT)])
        if with_cnt:
            pltpu.sync_copy(zc_hbm, cnt_sh.at[pl.ds(row0, RPT)])
            pltpu.sync_copy(ones_hbm, ones_v)
        plsc.subcore_barrier()

        for b in range(NB):
            pltpu.async_copy(xl_hbm.at[src_v.at[b]], rows_v.at[b], sem)

        @pl.loop(0, NCH, step=NB)
        def _edge_chunk(j):
            for b in range(NB):
                pltpu.make_async_copy(xl_hbm.at[pl.ds(0, C)],
                                      rows_v.at[b], sem).wait()
                pltpu.sync_copy(rows_v.at[b], acc_sh.at[dst_v.at[j + b]],
                                add=True)

                @pl.when(j + b + NB < NCH)
                def _():
                    pltpu.async_copy(xl_hbm.at[src_v.at[j + b + NB]],
                                     rows_v.at[b], sem)

                if with_cnt:
                    pltpu.sync_copy(ones_v, cnt_sh.at[dst_v.at[j + b]],
                                    add=True)

        plsc.subcore_barrier()
        pltpu.sync_copy(acc_sh.at[pl.ds(row0, RPT)],
                        s_out.at[pl.ds(c * NA + row0, RPT)])
        if with_cnt:
            pltpu.sync_copy(cnt_sh.at[pl.ds(row0, RPT)],
                            cnt_out.at[pl.ds(c * NA + row0, RPT)])

    return pl.kernel(body, out_type=tuple(out_type), mesh=mesh,
                     scratch_types=scratch,
                     compiler_params=pltpu.CompilerParams(
                         use_tc_tiling_on_sc=False))


def _tc_proj(x_ref, wl_ref, wr_ref, b_ref, xl_ref, r_ref):
    x = x_ref[...]
    xl_ref[:N, :] = jnp.dot(x, wl_ref[...], preferred_element_type=jnp.float32)
    xl_ref[N:, :] = jnp.zeros((NA - N, D), jnp.float32)
    r_ref[...] = (jnp.dot(x, wr_ref[...], preferred_element_type=jnp.float32)
                  + b_ref[...])


def _tc_first(s_ref, cnt_ref, r_ref, wl_ref, wr_ref, b_ref,
              h_ref, xl_ref, rn_ref, inv_ref):
    cnt = cnt_ref[:N, 0:1] + cnt_ref[NA:NA + N, 0:1]
    inv = 1.0 / jnp.maximum(cnt, 1.0)
    inv_ref[...] = inv
    ssum = s_ref[:N, :] + s_ref[NA:NA + N, :]
    h = jnp.maximum(ssum * inv + r_ref[...], 0.0)
    h_ref[...] = h
    xl_ref[:N, :] = jnp.dot(h, wl_ref[...], preferred_element_type=jnp.float32)
    xl_ref[N:, :] = jnp.zeros((NA - N, D), jnp.float32)
    rn_ref[...] = (jnp.dot(h, wr_ref[...], preferred_element_type=jnp.float32)
                   + b_ref[...])


def _tc_mid(s_ref, inv_ref, r_ref, res_ref, wl_ref, wr_ref, b_ref,
            h_ref, xl_ref, rn_ref):
    ssum = s_ref[:N, :] + s_ref[NA:NA + N, :]
    h = jnp.maximum(ssum * inv_ref[...] + r_ref[...] + res_ref[...], 0.0)
    h_ref[...] = h
    xl_ref[:N, :] = jnp.dot(h, wl_ref[...], preferred_element_type=jnp.float32)
    xl_ref[N:, :] = jnp.zeros((NA - N, D), jnp.float32)
    rn_ref[...] = (jnp.dot(h, wr_ref[...], preferred_element_type=jnp.float32)
                   + b_ref[...])


def _tc_final(s_ref, inv_ref, r_ref, out_ref):
    ssum = s_ref[:N, :] + s_ref[NA:NA + N, :]
    out_ref[...] = ssum * inv_ref[...] + r_ref[...]


def _call(body, out_shapes, *args):
    return pl.pallas_call(body, out_shape=out_shapes)(*args)


def kernel(x, edge_index, Wl0, bl0, Wr0, Wl1, bl1, Wr1, Wl2, bl2, Wr2,
           Wl3, bl3, Wr3):
    f32 = jnp.float32
    src = edge_index[0]
    dst = edge_index[1]
    pad = EPAD - E
    src_p = jnp.concatenate([src, jnp.zeros((pad,), jnp.int32)])
    dst_p = jnp.concatenate([dst, jnp.full((pad,), N, jnp.int32)])
    src_p = src_p.reshape(NW * NCH, C)
    dst_p = dst_p.reshape(NW * NCH, C)
    z32 = jnp.zeros((RPT, D), f32)
    z16 = jnp.zeros((RPT, CW), f32)
    ones = jnp.ones((C, CW), f32)
    mk = jax.ShapeDtypeStruct

    sc_cnt = _sc_scatter(True)
    sc = _sc_scatter(False)

    xl0, r0 = _call(_tc_proj, (mk((NA, D), f32), mk((N, D), f32)),
                    x, Wl0.T, Wr0.T, bl0.reshape(1, D))
    s0, cnt = sc_cnt(xl0, src_p, dst_p, z32, z16, ones)
    h0, xl1, r1, inv = _call(
        _tc_first, (mk((N, D), f32), mk((NA, D), f32), mk((N, D), f32),
                    mk((N, 1), f32)),
        s0, cnt, r0, Wl1.T, Wr1.T, bl1.reshape(1, D))
    (s1,) = sc(xl1, src_p, dst_p, z32)
    h1, xl2, r2 = _call(
        _tc_mid, (mk((N, D), f32), mk((NA, D), f32), mk((N, D), f32)),
        s1, inv, r1, h0, Wl2.T, Wr2.T, bl2.reshape(1, D))
    (s2,) = sc(xl2, src_p, dst_p, z32)
    h2, xl3, r3 = _call(
        _tc_mid, (mk((N, D), f32), mk((NA, D), f32), mk((N, D), f32)),
        s2, inv, r2, h1, Wl3.T, Wr3.T, bl3.reshape(1, D))
    (s3,) = sc(xl3, src_p, dst_p, z32)
    out = _call(_tc_final, mk((N, D), f32), s3, inv, r3)
    return out
